# baseline (speedup 1.0000x reference)
import jax
import jax.numpy as jnp
from jax import lax
from jax.experimental import pallas as pl
from jax.experimental.pallas import tpu as pltpu
from jax.experimental.pallas import tpu_sc as plsc

_NC, _NS = 2, 16
_NW = _NC * _NS
_ROWS = 4 * 4096
_D = 2048

_TC_ROWS = 9216
_SC_ROWS = _ROWS - _TC_ROWS
_TC_BLOCK = 1024
_SC_CH = 8


def _tc_body(x_ref, o_ref):
    o_ref[...] = x_ref[...]


def _sc_copy(x_hbm, o_hbm, buf0, buf1, sem0, sem1):
    wid = lax.axis_index("s") * _NC + lax.axis_index("c")
    rows_per_w = _SC_ROWS // _NW
    base = wid * rows_per_w
    n_chunks = rows_per_w // _SC_CH
    bufs = (buf0, buf1)
    sems = (sem0, sem1)

    def in_copy(ci, slot):
        return pltpu.make_async_copy(
            x_hbm.at[pl.ds(base + ci * _SC_CH, _SC_CH)], bufs[slot], sems[slot]
        )

    def out_copy(ci, slot):
        return pltpu.make_async_copy(
            bufs[slot], o_hbm.at[pl.ds(base + ci * _SC_CH, _SC_CH)], sems[slot]
        )

    in_copy(0, 0).start()

    def body(i, _):
        ci0 = 2 * i
        in_copy(ci0 + 1, 1).start()
        in_copy(ci0, 0).wait()
        out_copy(ci0, 0).start()
        out_copy(ci0, 0).wait()

        @pl.when(ci0 + 2 < n_chunks)
        def _():
            in_copy(ci0 + 2, 0).start()

        in_copy(ci0 + 1, 1).wait()
        out_copy(ci0 + 1, 1).start()
        out_copy(ci0 + 1, 1).wait()
        return 0

    lax.fori_loop(0, n_chunks // 2, body, 0)


def kernel(inputs, medians):
    del medians
    B, S, D = inputs.shape
    x = inputs.reshape(B * S, D)

    run_sc = pl.kernel(
        _sc_copy,
        out_type=jax.ShapeDtypeStruct((_SC_ROWS, D), jnp.float32),
        mesh=plsc.VectorSubcoreMesh(core_axis_name="c", subcore_axis_name="s"),
        scratch_types=[
            pltpu.VMEM((_SC_CH, _D), jnp.float32),
            pltpu.VMEM((_SC_CH, _D), jnp.float32),
            pltpu.SemaphoreType.DMA,
            pltpu.SemaphoreType.DMA,
        ],
    )
    out_sc = run_sc(x[_TC_ROWS:])

    out_tc = pl.pallas_call(
        _tc_body,
        grid=(_TC_ROWS // _TC_BLOCK,),
        in_specs=[pl.BlockSpec((_TC_BLOCK, D), lambda i: (i, 0))],
        out_specs=pl.BlockSpec((_TC_BLOCK, D), lambda i: (i, 0)),
        out_shape=jax.ShapeDtypeStruct((_TC_ROWS, D), inputs.dtype),
        compiler_params=pltpu.CompilerParams(
            dimension_semantics=("parallel",),
        ),
    )(x[:_TC_ROWS])

    return jnp.concatenate([out_tc, out_sc], axis=0).reshape(B, S, D)


# speedup vs baseline: 3.2710x; 3.2710x over previous
import jax
import jax.numpy as jnp
from jax.experimental import pallas as pl
from jax.experimental.pallas import tpu as pltpu

_ROWS = 4 * 4096
_D = 2048
_CH = 256
_NB = 8
_LEAD = 4
_NCHUNKS = _ROWS // _CH


def _dma_ring_body(x_ref, o_ref):
    def scoped(bufs, fsems, dsems):
        def fill(ci):
            s = ci % _NB
            return pltpu.make_async_copy(
                x_ref.at[pl.ds(ci * _CH, _CH)], bufs.at[s], fsems.at[s]
            )

        def drain(ci):
            s = ci % _NB
            return pltpu.make_async_copy(
                bufs.at[s], o_ref.at[pl.ds(ci * _CH, _CH)], dsems.at[s]
            )

        for i in range(_NCHUNKS + _LEAD):
            if i < _NCHUNKS:
                if i >= _NB:
                    drain(i - _NB).wait()
                fill(i).start()
            j = i - _LEAD
            if j >= 0:
                fill(j).wait()
                drain(j).start()
        for j in range(_NCHUNKS - _NB, _NCHUNKS):
            drain(j).wait()

    pl.run_scoped(
        scoped,
        pltpu.VMEM((_NB, _CH, _D), jnp.float32),
        pltpu.SemaphoreType.DMA((_NB,)),
        pltpu.SemaphoreType.DMA((_NB,)),
    )


def kernel(inputs, medians):
    del medians
    B, S, D = inputs.shape
    x = inputs.reshape(B * S, D)
    out = pl.pallas_call(
        _dma_ring_body,
        in_specs=[pl.BlockSpec(memory_space=pl.ANY)],
        out_specs=pl.BlockSpec(memory_space=pl.ANY),
        out_shape=jax.ShapeDtypeStruct((B * S, D), inputs.dtype),
    )(x)
    return out.reshape(B, S, D)
